# baseline (speedup 1.0000x reference)
import jax
import jax.numpy as jnp
from jax import lax
from jax.experimental import pallas as pl
from jax.experimental.pallas import tpu as pltpu
from jax.experimental.pallas import tpu_sc as plsc

N = 10000
E = 160000
D = 256
NPAD = 10240
NC = 2
NS = 16
HALF = N // NC
SROWS = 5120
TPT = SROWS // NS
NW = NC * NS
TPW = NPAD // NW
ACCR = TPW + 8
CH = 3200
NCHE = E // CH
CL = 2 * CH + 80
SB = 64
EPW = E // (NC * NS)
KPT = NPAD // NS

BR = 400
GRID = N // BR


def _mesh():
    return plsc.VectorSubcoreMesh(core_axis_name="c", subcore_axis_name="s")


_SC_PARAMS = pltpu.CompilerParams(needs_layout_passes=False)


def _deg_body(dst_hbm, out_hbm, dst_v, hist_v, part_v, outv_v, shared_v):
    c = lax.axis_index("c")
    s = lax.axis_index("s")
    w = s * NC + c
    base = pl.multiple_of(w * EPW, 8)
    pltpu.sync_copy(dst_hbm.at[pl.ds(base, EPW)], dst_v)

    zeros16 = jnp.zeros((16,), jnp.float32)
    ones16 = jnp.ones((16,), jnp.float32)

    @pl.loop(0, NPAD // 16)
    def _zero(i):
        hist_v[pl.ds(i * 16, 16)] = zeros16

    @pl.loop(0, EPW // 16)
    def _hist(i):
        idx = dst_v[pl.ds(i * 16, 16)]
        plsc.addupdate_scatter(hist_v, [idx], ones16)

    if EPW % 16:
        lane = lax.iota(jnp.int32, 16)
        idx = dst_v[pl.ds(EPW - 16, 16)]
        plsc.addupdate_scatter(hist_v, [idx], ones16, mask=lane >= (16 - EPW % 16))

    pltpu.sync_copy(hist_v, shared_v.at[s])
    plsc.subcore_barrier()

    kbase = pl.multiple_of(s * KPT, 8)
    for r in range(NS):
        pltpu.sync_copy(shared_v.at[r, pl.ds(kbase, KPT)], part_v.at[r])

    @pl.loop(0, KPT // 16)
    def _reduce(j):
        acc = part_v[0, pl.ds(j * 16, 16)]
        for r in range(1, NS):
            acc = acc + part_v[r, pl.ds(j * 16, 16)]
        outv_v[pl.ds(j * 16, 16)] = acc

    pltpu.sync_copy(outv_v, out_hbm.at[c, pl.ds(kbase, KPT)])


def _deg_call(dst):
    return pl.kernel(
        _deg_body,
        out_type=jax.ShapeDtypeStruct((NC, NPAD), jnp.float32),
        mesh=_mesh(),
        compiler_params=_SC_PARAMS,
        scratch_types=[
            pltpu.VMEM((EPW,), jnp.int32),
            pltpu.VMEM((NPAD,), jnp.float32),
            pltpu.VMEM((NS, KPT), jnp.float32),
            pltpu.VMEM((KPT,), jnp.float32),
            pltpu.VMEM_SHARED((NS, NPAD), jnp.float32),
        ],
    )(dst)


def _edge_body(hp_hbm, src_hbm, dst_hbm, zeros_hbm, acc_hbm,
               sb0, db0, sb1, db1, pklist, srcbuf, locbuf, rows, acc,
               semA, semB, semG):
    c = lax.axis_index("c")
    s = lax.axis_index("s")
    w = s * NC + c
    base = w * TPW

    iota16 = lax.iota(jnp.int32, 16)
    zeros16i = jnp.zeros((16,), jnp.int32)
    trash16 = TPW + (iota16 & 7)

    pltpu.sync_copy(zeros_hbm, acc)

    def stage_start(ch, sbuf, dbuf, sem):
        eoff = pl.multiple_of(ch * CH, 8)
        pltpu.async_copy(src_hbm.at[pl.ds(eoff, CH)], sbuf, sem)
        pltpu.async_copy(dst_hbm.at[pl.ds(eoff, CH)], dbuf, sem)

    def stage_wait(sbuf, dbuf, sem):
        pltpu.make_async_copy(src_hbm.at[pl.ds(0, CH)], sbuf, sem).wait()
        pltpu.make_async_copy(dst_hbm.at[pl.ds(0, CH)], dbuf, sem).wait()

    def scan_chunk(sbuf, dbuf, cnt0):
        def scan(i, cnt):
            d = dbuf[pl.ds(i * 16, 16)]
            sv = sbuf[pl.ds(i * 16, 16)]
            loc = d - base
            mask = (loc >= 0) & (loc < TPW)
            locs = jnp.where(mask, loc, TPW + (d & 7))
            packed = jnp.bitwise_or(jnp.left_shift(sv, 9), locs)
            pklist[pl.ds(i * 16, 16)] = packed
            return cnt

        return pl.loop(0, CH // 16, init_carry=cnt0, unroll=4)(scan) + 200

    def flush(m):
        for kpad in range(SB // 16):
            pklist[pl.ds(m + kpad * 16, 16)] = trash16

        nb = jnp.right_shift(m + (SB - 1), SB.bit_length() - 1)

        @pl.loop(0, nb)
        def _batch(b):
            boff = pl.multiple_of(b * SB, 8)
            for g in range(SB // 16):
                packed = pklist[pl.ds(boff + g * 16, 16)]
                srcbuf[pl.ds(g * 16, 16)] = jnp.right_shift(packed, 9)
                locbuf[pl.ds(g * 16, 16)] = packed & 511
            pltpu.async_copy(hp_hbm.at[srcbuf], rows, semG)
            pltpu.make_async_copy(hp_hbm.at[srcbuf], rows, semG).wait()
            for g in range(SB // 16):
                locv = locbuf[pl.ds(g * 16, 16)]
                rowv = iota16 + g * 16

                @pl.loop(0, D // 16, unroll=4)
                def _cb(cb):
                    for colr in range(16):
                        cv = jnp.full((16,), cb * 16 + colr, jnp.int32)
                        vals = plsc.load_gather(rows, [rowv, cv])
                        plsc.addupdate_scatter(acc, [locv, cv], vals)

    stage_start(0, sb0, db0, semA)

    @pl.loop(0, NCHE // 2)
    def _pair(p):
        ch0 = p * 2
        stage_wait(sb0, db0, semA)
        stage_start(ch0 + 1, sb1, db1, semB)
        m0 = scan_chunk(sb0, db0, jnp.int32(0))
        stage_wait(sb1, db1, semB)

        @pl.when(ch0 + 2 < NCHE)
        def _():
            stage_start(ch0 + 2, sb0, db0, semA)

        m1 = scan_chunk(sb1, db1, m0)
        flush(m1)

    @pl.when(w < NW - 1)
    def _full():
        pltpu.sync_copy(acc.at[pl.ds(0, TPW)],
                        acc_hbm.at[pl.ds(base, TPW)])

    last = N - (NW - 1) * TPW

    @pl.when(w == NW - 1)
    def _last():
        pltpu.sync_copy(acc.at[pl.ds(0, last)],
                        acc_hbm.at[pl.ds(base, last)])


def _edge_call(hp, src, dst, zeros_acc):
    return pl.kernel(
        _edge_body,
        out_type=jax.ShapeDtypeStruct((N, D), jnp.float32),
        mesh=_mesh(),
        compiler_params=_SC_PARAMS,
        scratch_types=[
            pltpu.VMEM((CH,), jnp.int32),
            pltpu.VMEM((CH,), jnp.int32),
            pltpu.VMEM((CH,), jnp.int32),
            pltpu.VMEM((CH,), jnp.int32),
            pltpu.VMEM((CL,), jnp.int32),
            pltpu.VMEM((SB,), jnp.int32),
            pltpu.VMEM((SB,), jnp.int32),
            pltpu.VMEM((SB, D), jnp.float32),
            pltpu.VMEM((ACCR, D), jnp.float32),
            pltpu.SemaphoreType.DMA,
            pltpu.SemaphoreType.DMA,
            pltpu.SemaphoreType.DMA,
        ],
    )(hp, src, dst, zeros_acc)


def _dis_body(deg_ref, o_ref):
    d = deg_ref[0, :] + deg_ref[1, :] + 1.0
    o_ref[0, :] = lax.rsqrt(d)


def _dis_call(deg2):
    return pl.pallas_call(
        _dis_body,
        out_shape=jax.ShapeDtypeStruct((1, NPAD), jnp.float32),
    )(deg2)


def _mm_in_body(x_ref, w_ref, b_ref, dis_ref, o_ref):
    h = jnp.dot(x_ref[...], w_ref[...], preferred_element_type=jnp.float32)
    h = jnp.maximum(h + b_ref[...], 0.0)
    o_ref[...] = h * dis_ref[...]


def _mm_in(x, W, b2, dis_col):
    return pl.pallas_call(
        _mm_in_body,
        grid=(GRID,),
        in_specs=[pl.BlockSpec((BR, D), lambda i: (i, 0)),
                  pl.BlockSpec((D, D), lambda i: (0, 0)),
                  pl.BlockSpec((1, D), lambda i: (0, 0)),
                  pl.BlockSpec((BR, 1), lambda i: (i, 0))],
        out_specs=pl.BlockSpec((BR, D), lambda i: (i, 0)),
        out_shape=jax.ShapeDtypeStruct((N, D), jnp.float32),
    )(x, W, b2, dis_col)


def _layer_body(acc_ref, hp_ref, dis_ref, w_ref, b_ref, o_ref):
    dis = dis_ref[...]
    g = (acc_ref[...] + hp_ref[...]) * dis
    h = jnp.dot(g, w_ref[...], preferred_element_type=jnp.float32)
    h = jnp.maximum(h + b_ref[...], 0.0)
    o_ref[...] = h * dis


def _layer(acc, hp, dis_col, W, b2):
    return pl.pallas_call(
        _layer_body,
        grid=(GRID,),
        in_specs=[pl.BlockSpec((BR, D), lambda i: (i, 0)),
                  pl.BlockSpec((BR, D), lambda i: (i, 0)),
                  pl.BlockSpec((BR, 1), lambda i: (i, 0)),
                  pl.BlockSpec((D, D), lambda i: (0, 0)),
                  pl.BlockSpec((1, D), lambda i: (0, 0))],
        out_specs=pl.BlockSpec((BR, D), lambda i: (i, 0)),
        out_shape=jax.ShapeDtypeStruct((N, D), jnp.float32),
    )(acc, hp, dis_col, W, b2)


def _final_body(acc_ref, hp_ref, dis_ref, w_ref, b_ref, wh_ref, bh_ref, o_ref):
    dis = dis_ref[...]
    g = (acc_ref[...] + hp_ref[...]) * dis
    h = jnp.dot(g, w_ref[...], preferred_element_type=jnp.float32)
    h = jnp.maximum(h + b_ref[...], 0.0)
    t = jnp.dot(h, wh_ref[...], preferred_element_type=jnp.float32) + bh_ref[...]
    lane = lax.broadcasted_iota(jnp.int32, t.shape, 1)
    is_c = lane < 3
    m = jnp.max(jnp.where(is_c, t, -1e30), axis=1, keepdims=True)
    e = jnp.where(is_c, jnp.exp(t - m), 0.0)
    cls = e / jnp.sum(e, axis=1, keepdims=True)
    score = 1.0 / (1.0 + jnp.exp(-t))
    o_ref[...] = jnp.where(is_c, cls, jnp.where(lane == 3, score, 0.0))


def _final(acc, hp, dis_col, W, b2, Wh, bh):
    return pl.pallas_call(
        _final_body,
        grid=(GRID,),
        in_specs=[pl.BlockSpec((BR, D), lambda i: (i, 0)),
                  pl.BlockSpec((BR, D), lambda i: (i, 0)),
                  pl.BlockSpec((BR, 1), lambda i: (i, 0)),
                  pl.BlockSpec((D, D), lambda i: (0, 0)),
                  pl.BlockSpec((1, D), lambda i: (0, 0)),
                  pl.BlockSpec((D, 128), lambda i: (0, 0)),
                  pl.BlockSpec((1, 128), lambda i: (0, 0))],
        out_specs=pl.BlockSpec((BR, 128), lambda i: (i, 0)),
        out_shape=jax.ShapeDtypeStruct((N, 128), jnp.float32),
    )(acc, hp, dis_col, W, b2, Wh, bh)


def kernel(x, edge_index, W_in, b_in, W1, b1, W2, b2, Wc, bc, Ws, bs):
    src = edge_index[0]
    dst = edge_index[1]

    deg2 = _deg_call(dst)
    dis_row = _dis_call(deg2)
    dis_col = dis_row.reshape(NPAD, 1)[:N]

    zeros_acc = jnp.zeros((ACCR, D), jnp.float32)
    h0p = _mm_in(x, W_in, b_in.reshape(1, D), dis_col)
    acc1 = _edge_call(h0p, src, dst, zeros_acc)
    h1p = _layer(acc1, h0p, dis_col, W1, b1.reshape(1, D))
    acc2 = _edge_call(h1p, src, dst, zeros_acc)

    Wh = jnp.zeros((D, 128), jnp.float32).at[:, :3].set(Wc).at[:, 3:4].set(Ws)
    bh = jnp.zeros((1, 128), jnp.float32).at[0, :3].set(bc).at[0, 3].set(bs[0])
    out128 = _final(acc2, h1p, dis_col, W2, b2.reshape(1, D), Wh, bh)
    return out128[:, :3], out128[:, 3:4]


# speedup vs baseline: 2.2534x; 2.2534x over previous
import jax
import jax.numpy as jnp
from jax import lax
from jax.experimental import pallas as pl
from jax.experimental.pallas import tpu as pltpu
from jax.experimental.pallas import tpu_sc as plsc

N = 10000
E = 160000
D = 256
NPAD = 10240
NC = 2
NS = 16
HALF = N // NC
SROWS = 5120
TPT = SROWS // NS
NW = NC * NS
TPW = NPAD // NW
ACCR = TPW + 8
CH = 1600
NCHE = E // CH
CL = 6480
SB = 64
EPW = E // (NC * NS)
KPT = NPAD // NS

BR = 400
GRID = N // BR


def _mesh():
    return plsc.VectorSubcoreMesh(core_axis_name="c", subcore_axis_name="s")


_SC_PARAMS = pltpu.CompilerParams(needs_layout_passes=False)


def _deg_body(dst_hbm, out_hbm, dst_v, hist_v, part_v, outv_v, shared_v):
    c = lax.axis_index("c")
    s = lax.axis_index("s")
    w = s * NC + c
    base = pl.multiple_of(w * EPW, 8)
    pltpu.sync_copy(dst_hbm.at[pl.ds(base, EPW)], dst_v)

    zeros16 = jnp.zeros((16,), jnp.float32)
    ones16 = jnp.ones((16,), jnp.float32)

    @pl.loop(0, NPAD // 16)
    def _zero(i):
        hist_v[pl.ds(i * 16, 16)] = zeros16

    @pl.loop(0, EPW // 16)
    def _hist(i):
        idx = dst_v[pl.ds(i * 16, 16)]
        plsc.addupdate_scatter(hist_v, [idx], ones16)

    if EPW % 16:
        lane = lax.iota(jnp.int32, 16)
        idx = dst_v[pl.ds(EPW - 16, 16)]
        plsc.addupdate_scatter(hist_v, [idx], ones16, mask=lane >= (16 - EPW % 16))

    pltpu.sync_copy(hist_v, shared_v.at[s])
    plsc.subcore_barrier()

    kbase = pl.multiple_of(s * KPT, 8)
    for r in range(NS):
        pltpu.sync_copy(shared_v.at[r, pl.ds(kbase, KPT)], part_v.at[r])

    @pl.loop(0, KPT // 16)
    def _reduce(j):
        acc = part_v[0, pl.ds(j * 16, 16)]
        for r in range(1, NS):
            acc = acc + part_v[r, pl.ds(j * 16, 16)]
        outv_v[pl.ds(j * 16, 16)] = acc

    pltpu.sync_copy(outv_v, out_hbm.at[c, pl.ds(kbase, KPT)])


def _deg_call(dst):
    return pl.kernel(
        _deg_body,
        out_type=jax.ShapeDtypeStruct((NC, NPAD), jnp.float32),
        mesh=_mesh(),
        compiler_params=_SC_PARAMS,
        scratch_types=[
            pltpu.VMEM((EPW,), jnp.int32),
            pltpu.VMEM((NPAD,), jnp.float32),
            pltpu.VMEM((NS, KPT), jnp.float32),
            pltpu.VMEM((KPT,), jnp.float32),
            pltpu.VMEM_SHARED((NS, NPAD), jnp.float32),
        ],
    )(dst)


def _edge_body(hp_hbm, src_hbm, dst_hbm, zeros_hbm, acc_hbm,
               sb0, db0, sb1, db1, pklist, srcbuf, locbuf, rows, acc,
               semA, semB, semG):
    c = lax.axis_index("c")
    s = lax.axis_index("s")
    w = s * NC + c
    base = w * TPW

    iota16 = lax.iota(jnp.int32, 16)
    zeros16i = jnp.zeros((16,), jnp.int32)
    trash16 = TPW + (iota16 & 7)

    pltpu.sync_copy(zeros_hbm, acc)

    def stage_start(ch, sbuf, dbuf, sem):
        eoff = pl.multiple_of(ch * CH, 8)
        pltpu.async_copy(src_hbm.at[pl.ds(eoff, CH)], sbuf, sem)
        pltpu.async_copy(dst_hbm.at[pl.ds(eoff, CH)], dbuf, sem)

    def stage_wait(sbuf, dbuf, sem):
        pltpu.make_async_copy(src_hbm.at[pl.ds(0, CH)], sbuf, sem).wait()
        pltpu.make_async_copy(dst_hbm.at[pl.ds(0, CH)], dbuf, sem).wait()

    def scan_chunk(sbuf, dbuf, cnt0):
        def scan(i, cnt):
            d = dbuf[pl.ds(i * 16, 16)]
            sv = sbuf[pl.ds(i * 16, 16)]
            loc = d - base
            mask = (loc >= 0) & (loc < TPW)
            packed = jnp.bitwise_or(jnp.left_shift(sv, 9), loc & 511)
            plsc.store_compressed(pklist.at[pl.ds(cnt, 16)], packed, mask=mask)
            pc = plsc.all_reduce_population_count(mask)
            return cnt + pc[0]

        return pl.loop(0, CH // 16, init_carry=cnt0, unroll=4)(scan)

    def unpack(b, par):
        boff = b * SB
        poff = par * SB
        for g in range(SB // 16):
            packed = pklist[pl.ds(boff + g * 16, 16)]
            srcbuf[pl.ds(poff + g * 16, 16)] = jnp.right_shift(packed, 9)
            locbuf[pl.ds(poff + g * 16, 16)] = packed & 511

    def gdesc(par):
        idxs = srcbuf.at[pl.ds(par * SB, SB)]
        return pltpu.make_async_copy(hp_hbm.at[idxs], rows.at[par], semG.at[par])

    def flush(m):
        for kpad in range(SB // 16):
            pklist[pl.ds(m + kpad * 16, 16)] = trash16

        nb = jnp.right_shift(m + (SB - 1), SB.bit_length() - 1)

        @pl.when(nb > 0)
        def _prime():
            unpack(0, 0)
            gdesc(0).start()

        @pl.loop(0, nb)
        def _batch(b):
            par = b & 1
            npar = 1 - par

            @pl.when(b + 1 < nb)
            def _():
                unpack(b + 1, npar)
                gdesc(npar).start()

            gdesc(par).wait()
            parv = jnp.full((16,), par, jnp.int32)
            for g in range(SB // 16):
                locv = locbuf[pl.ds(par * SB + g * 16, 16)]
                rowv = iota16 + g * 16

                @pl.loop(0, D // 16, unroll=2)
                def _cb(cb):
                    for colr in range(16):
                        cv = jnp.full((16,), cb * 16 + colr, jnp.int32)
                        vals = plsc.load_gather(rows, [parv, rowv, cv])
                        plsc.addupdate_scatter(acc, [locv, cv], vals)

    stage_start(0, sb0, db0, semA)

    def maybe_flush(m):
        cond = m > CL - 2 * CH

        @pl.when(cond)
        def _():
            flush(m)

        return jnp.where(cond, jnp.int32(0), m)

    @pl.loop(0, NCHE // 2, init_carry=jnp.int32(0))
    def _pair(p, m):
        ch0 = p * 2
        stage_wait(sb0, db0, semA)
        stage_start(ch0 + 1, sb1, db1, semB)
        m = scan_chunk(sb0, db0, m)
        stage_wait(sb1, db1, semB)

        @pl.when(ch0 + 2 < NCHE)
        def _():
            stage_start(ch0 + 2, sb0, db0, semA)

        m = scan_chunk(sb1, db1, m)
        return maybe_flush(m)

    mfin = _pair
    flush(mfin)

    @pl.when(w < NW - 1)
    def _full():
        pltpu.sync_copy(acc.at[pl.ds(0, TPW)],
                        acc_hbm.at[pl.ds(base, TPW)])

    last = N - (NW - 1) * TPW

    @pl.when(w == NW - 1)
    def _last():
        pltpu.sync_copy(acc.at[pl.ds(0, last)],
                        acc_hbm.at[pl.ds(base, last)])


def _edge_call(hp, src, dst, zeros_acc):
    return pl.kernel(
        _edge_body,
        out_type=jax.ShapeDtypeStruct((N, D), jnp.float32),
        mesh=_mesh(),
        compiler_params=_SC_PARAMS,
        scratch_types=[
            pltpu.VMEM((CH,), jnp.int32),
            pltpu.VMEM((CH,), jnp.int32),
            pltpu.VMEM((CH,), jnp.int32),
            pltpu.VMEM((CH,), jnp.int32),
            pltpu.VMEM((CL,), jnp.int32),
            pltpu.VMEM((2 * SB,), jnp.int32),
            pltpu.VMEM((2 * SB,), jnp.int32),
            pltpu.VMEM((2, SB, D), jnp.float32),
            pltpu.VMEM((ACCR, D), jnp.float32),
            pltpu.SemaphoreType.DMA,
            pltpu.SemaphoreType.DMA,
            pltpu.SemaphoreType.DMA((2,)),
        ],
    )(hp, src, dst, zeros_acc)


def _dis_body(deg_ref, o_ref):
    d = deg_ref[0, :] + deg_ref[1, :] + 1.0
    o_ref[0, :] = lax.rsqrt(d)


def _dis_call(deg2):
    return pl.pallas_call(
        _dis_body,
        out_shape=jax.ShapeDtypeStruct((1, NPAD), jnp.float32),
    )(deg2)


def _mm_in_body(x_ref, w_ref, b_ref, dis_ref, o_ref):
    h = jnp.dot(x_ref[...], w_ref[...], preferred_element_type=jnp.float32)
    h = jnp.maximum(h + b_ref[...], 0.0)
    o_ref[...] = h * dis_ref[...]


def _mm_in(x, W, b2, dis_col):
    return pl.pallas_call(
        _mm_in_body,
        grid=(GRID,),
        in_specs=[pl.BlockSpec((BR, D), lambda i: (i, 0)),
                  pl.BlockSpec((D, D), lambda i: (0, 0)),
                  pl.BlockSpec((1, D), lambda i: (0, 0)),
                  pl.BlockSpec((BR, 1), lambda i: (i, 0))],
        out_specs=pl.BlockSpec((BR, D), lambda i: (i, 0)),
        out_shape=jax.ShapeDtypeStruct((N, D), jnp.float32),
    )(x, W, b2, dis_col)


def _layer_body(acc_ref, hp_ref, dis_ref, w_ref, b_ref, o_ref):
    dis = dis_ref[...]
    g = (acc_ref[...] + hp_ref[...]) * dis
    h = jnp.dot(g, w_ref[...], preferred_element_type=jnp.float32)
    h = jnp.maximum(h + b_ref[...], 0.0)
    o_ref[...] = h * dis


def _layer(acc, hp, dis_col, W, b2):
    return pl.pallas_call(
        _layer_body,
        grid=(GRID,),
        in_specs=[pl.BlockSpec((BR, D), lambda i: (i, 0)),
                  pl.BlockSpec((BR, D), lambda i: (i, 0)),
                  pl.BlockSpec((BR, 1), lambda i: (i, 0)),
                  pl.BlockSpec((D, D), lambda i: (0, 0)),
                  pl.BlockSpec((1, D), lambda i: (0, 0))],
        out_specs=pl.BlockSpec((BR, D), lambda i: (i, 0)),
        out_shape=jax.ShapeDtypeStruct((N, D), jnp.float32),
    )(acc, hp, dis_col, W, b2)


def _final_body(acc_ref, hp_ref, dis_ref, w_ref, b_ref, wh_ref, bh_ref, o_ref):
    dis = dis_ref[...]
    g = (acc_ref[...] + hp_ref[...]) * dis
    h = jnp.dot(g, w_ref[...], preferred_element_type=jnp.float32)
    h = jnp.maximum(h + b_ref[...], 0.0)
    t = jnp.dot(h, wh_ref[...], preferred_element_type=jnp.float32) + bh_ref[...]
    lane = lax.broadcasted_iota(jnp.int32, t.shape, 1)
    is_c = lane < 3
    m = jnp.max(jnp.where(is_c, t, -1e30), axis=1, keepdims=True)
    e = jnp.where(is_c, jnp.exp(t - m), 0.0)
    cls = e / jnp.sum(e, axis=1, keepdims=True)
    score = 1.0 / (1.0 + jnp.exp(-t))
    o_ref[...] = jnp.where(is_c, cls, jnp.where(lane == 3, score, 0.0))


def _final(acc, hp, dis_col, W, b2, Wh, bh):
    return pl.pallas_call(
        _final_body,
        grid=(GRID,),
        in_specs=[pl.BlockSpec((BR, D), lambda i: (i, 0)),
                  pl.BlockSpec((BR, D), lambda i: (i, 0)),
                  pl.BlockSpec((BR, 1), lambda i: (i, 0)),
                  pl.BlockSpec((D, D), lambda i: (0, 0)),
                  pl.BlockSpec((1, D), lambda i: (0, 0)),
                  pl.BlockSpec((D, 128), lambda i: (0, 0)),
                  pl.BlockSpec((1, 128), lambda i: (0, 0))],
        out_specs=pl.BlockSpec((BR, 128), lambda i: (i, 0)),
        out_shape=jax.ShapeDtypeStruct((N, 128), jnp.float32),
    )(acc, hp, dis_col, W, b2, Wh, bh)


def kernel(x, edge_index, W_in, b_in, W1, b1, W2, b2, Wc, bc, Ws, bs):
    src = edge_index[0]
    dst = edge_index[1]

    deg2 = _deg_call(dst)
    dis_row = _dis_call(deg2)
    dis_col = dis_row.reshape(NPAD, 1)[:N]

    zeros_acc = jnp.zeros((ACCR, D), jnp.float32)
    h0p = _mm_in(x, W_in, b_in.reshape(1, D), dis_col)
    acc1 = _edge_call(h0p, src, dst, zeros_acc)
    h1p = _layer(acc1, h0p, dis_col, W1, b1.reshape(1, D))
    acc2 = _edge_call(h1p, src, dst, zeros_acc)

    Wh = jnp.zeros((D, 128), jnp.float32).at[:, :3].set(Wc).at[:, 3:4].set(Ws)
    bh = jnp.zeros((1, 128), jnp.float32).at[0, :3].set(bc).at[0, 3].set(bs[0])
    out128 = _final(acc2, h1p, dis_col, W2, b2.reshape(1, D), Wh, bh)
    return out128[:, :3], out128[:, 3:4]
